# baseline (speedup 1.0000x reference)
import functools

import jax
import jax.numpy as jnp
from jax import lax
from jax.experimental import pallas as pl
from jax.experimental.pallas import tpu as pltpu
from jax.experimental.pallas import tpu_sc as plsc

N_N = 10000
C_C = 128
E_E = 320000
EPS_V = 0.1

NC, NS, L = 2, 16, 16
NW = NC * NS
NP = 10240
EPW = E_E // NW
B = 80
NCH = EPW // B
MAIN = (NCH // 6) * 6
MB = 80
NCHM = EPW // MB
MAINM = (NCHM // 6) * 6

_mesh = plsc.VectorSubcoreMesh(
    core_axis_name="c", subcore_axis_name="s", num_cores=NC, num_subcores=NS)


@functools.partial(
    pl.kernel,
    out_type=jax.ShapeDtypeStruct((NC, NP), jnp.float32),
    mesh=_mesh,
    scratch_types=[
        pltpu.VMEM((6, B), jnp.int32),
        pltpu.VMEM((6, B), jnp.int32),
        pltpu.VMEM((3, B), jnp.float32),
        pltpu.VMEM((B,), jnp.float32),
        pltpu.VMEM((NP // NS,), jnp.float32),
        pltpu.VMEM_SHARED((NP,), jnp.float32),
        pltpu.SemaphoreType.DMA,
        pltpu.SemaphoreType.DMA,
        pltpu.SemaphoreType.DMA,
    ],
    compiler_params=pltpu.CompilerParams(needs_layout_passes=False),
)
def _deg_kernel(erow_ref, ecol_ref, zb_ref, out_ref, rbuf, cbuf, vbuf, zb,
                bounce, deg_sh, isem, ssem0, ssem1):
    cid = lax.axis_index("c")
    sid = lax.axis_index("s")
    w = sid * NC + cid
    base = w * EPW

    pltpu.sync_copy(zb_ref, zb)
    for j in range(8):
        pltpu.sync_copy(zb, deg_sh.at[pl.ds((sid + j * NS) * B, B)])
    plsc.subcore_barrier()

    def idx_start(off_chunks, s6):
        off = pl.multiple_of(base + off_chunks * B, 8)
        pltpu.async_copy(erow_ref.at[pl.ds(off, B)], rbuf.at[s6], isem)
        pltpu.async_copy(ecol_ref.at[pl.ds(off, B)], cbuf.at[s6], isem)

    def idx_wait():
        pltpu.make_async_copy(erow_ref.at[pl.ds(0, B)], rbuf.at[0],
                              isem).wait()
        pltpu.make_async_copy(ecol_ref.at[pl.ds(0, B)], cbuf.at[0],
                              isem).wait()

    def scatter_start(s6, s3, sem):
        pltpu.async_copy(vbuf.at[s3], deg_sh.at[cbuf.at[s6]], sem, add=True)

    def scatter_wait(sem):
        pltpu.make_async_copy(vbuf.at[0], deg_sh.at[cbuf.at[0]], sem).wait()

    def vals(s6, s3):
        for i in range(B // L):
            r = rbuf[s6, pl.ds(i * L, L)]
            c = cbuf[s6, pl.ds(i * L, L)]
            vbuf[s3, pl.ds(i * L, L)] = jnp.where(r != c, 1.0, 0.0)

    idx_start(0, 0)
    idx_wait()
    idx_start(1, 1)

    def body(k6, carry):
        kb = k6 * 6
        for t in range(6):
            kk = kb + t
            idx_wait()
            idx_start(kk + 2, (t + 2) % 6)
            vals(t, t % 3)
            sem = ssem0 if t % 2 == 0 else ssem1

            @pl.when(kk >= 2)
            def _():
                scatter_wait(sem)

            scatter_start(t, t % 3, sem)
        return carry

    lax.fori_loop(0, MAIN // 6, body, 0)
    for kk in range(MAIN, NCH):
        if kk + 1 < NCH:
            idx_wait()
        if kk + 2 < NCH:
            idx_start(kk + 2, (kk + 2) % 6)
        vals(kk % 6, kk % 3)
        sem = ssem0 if kk % 2 == 0 else ssem1
        scatter_wait(sem)
        scatter_start(kk % 6, kk % 3, sem)
    scatter_wait(ssem1)
    scatter_wait(ssem0)

    plsc.subcore_barrier()
    rpt = NP // NS
    pltpu.sync_copy(deg_sh.at[pl.ds(sid * rpt, rpt)], bounce)
    pltpu.sync_copy(bounce, out_ref.at[cid, pl.ds(sid * rpt, rpt)])


def _alar_body(x_ref, wl_ref, wr_ref, al_ref, ar_ref, pk_ref):
    x = x_ref[...]
    al = lax.dot_general(x, wl_ref[...], (((1,), (1,)), ((), ())))
    ar = lax.dot_general(x, wr_ref[...], (((1,), (1,)), ((), ())))
    al_ref[...] = al
    ar_ref[...] = ar
    ab = lax.bitcast_convert_type(al, jnp.int32)
    lo = ((ab + 0x7FFF + ((ab >> 16) & 1)) >> 16) & 0xFFFF
    bb = lax.bitcast_convert_type(ar, jnp.int32)
    hi = (bb + 0x7FFF + ((bb >> 16) & 1)) & jnp.int32(-65536)
    pk_ref[...] = hi | lo


_alar_call = pl.pallas_call(
    _alar_body,
    out_shape=(
        jax.ShapeDtypeStruct((N_N, 1), jnp.float32),
        jax.ShapeDtypeStruct((N_N, 1), jnp.float32),
        jax.ShapeDtypeStruct((N_N, 1), jnp.int32),
    ),
)


def _ds_body(x_ref, degp_ref, dinv_ref, y_ref):
    dinv = lax.rsqrt(degp_ref[0] + degp_ref[1] + 1.0)
    dinv_ref[...] = dinv
    y_ref[...] = x_ref[...] * dinv


_ds_call = pl.pallas_call(
    _ds_body,
    out_shape=(
        jax.ShapeDtypeStruct((N_N, 1), jnp.float32),
        jax.ShapeDtypeStruct((N_N, C_C), jnp.float32),
    ),
)


@functools.partial(
    pl.kernel,
    out_type=jax.ShapeDtypeStruct((NC, N_N, C_C), jnp.float32),
    mesh=_mesh,
    scratch_types=[
        pltpu.VMEM((N_N,), jnp.int32),
        pltpu.VMEM((6, MB), jnp.int32),
        pltpu.VMEM((6, MB), jnp.int32),
        pltpu.VMEM((3, MB, C_C), jnp.float32),
        pltpu.VMEM_SHARED((N_N, C_C), jnp.float32),
        pltpu.SemaphoreType.DMA,
        pltpu.SemaphoreType.DMA,
        pltpu.SemaphoreType.DMA,
    ],
    compiler_params=pltpu.CompilerParams(needs_layout_passes=False),
)
def _msg_kernel(erow_ref, ecol_ref, y_ref, pk_ref, zb_ref,
                out_ref, pkv, rbuf, cbuf, rows3, acc_sh,
                isem, gsem, ssem):
    cid = lax.axis_index("c")
    sid = lax.axis_index("s")
    w = sid * NC + cid
    base = w * EPW

    pltpu.sync_copy(pk_ref, pkv)
    pltpu.sync_copy(zb_ref, rows3.at[0])
    for j in range(8):
        ci = sid + j * NS

        @pl.when(ci < N_N // MB)
        def _():
            pltpu.sync_copy(rows3.at[0], acc_sh.at[pl.ds(ci * MB, MB), :])

    plsc.subcore_barrier()

    def idx_start(off_chunks, s6):
        off = pl.multiple_of(base + off_chunks * MB, 8)
        pltpu.async_copy(erow_ref.at[pl.ds(off, MB)], rbuf.at[s6], isem)
        pltpu.async_copy(ecol_ref.at[pl.ds(off, MB)], cbuf.at[s6], isem)

    def idx_wait():
        pltpu.make_async_copy(erow_ref.at[pl.ds(0, MB)], rbuf.at[0],
                              isem).wait()
        pltpu.make_async_copy(ecol_ref.at[pl.ds(0, MB)], cbuf.at[0],
                              isem).wait()

    def gather_start(s6, s3):
        pltpu.async_copy(y_ref.at[rbuf.at[s6]], rows3.at[s3], gsem)

    def gather_wait():
        pltpu.make_async_copy(y_ref.at[rbuf.at[0]], rows3.at[0], gsem).wait()

    def scatter_start(s6, s3):
        pltpu.async_copy(rows3.at[s3], acc_sh.at[cbuf.at[s6]], ssem, add=True)

    def scatter_wait():
        pltpu.make_async_copy(rows3.at[0], acc_sh.at[cbuf.at[0]],
                              ssem).wait()

    def compute(s6, s3):
        def grp(i, carry):
            r = rbuf[s6, pl.ds(i * L, L)]
            c = cbuf[s6, pl.ds(i * L, L)]
            pr = plsc.load_gather(pkv, [r])
            pc = plsc.load_gather(pkv, [c])
            af = plsc.bitcast(pr << 16, jnp.float32)
            bf = plsc.bitcast(pc & jnp.int32(-65536), jnp.float32)
            t = 1.0 - 2.0 / (jnp.exp(2.0 * (af + bf)) + 1.0)
            cv = jnp.where(r != c, t, 0.0)
            for e in range(L):
                cs = cv[e]
                row = i * L + e
                for j in range(C_C // L):
                    rows3[s3, row, pl.ds(j * L, L)] = (
                        rows3[s3, row, pl.ds(j * L, L)] * cs)
            return carry

        lax.fori_loop(0, MB // L, grp, 0)

    idx_start(0, 0)
    idx_wait()
    gather_start(0, 0)
    idx_start(1, 1)

    def body(k6, carry):
        kb = k6 * 6
        for t in range(6):
            kk = kb + t
            gather_wait()
            idx_wait()
            gather_start((t + 1) % 6, (t + 1) % 3)
            idx_start(kk + 2, (t + 2) % 6)
            compute(t, t % 3)

            @pl.when(kk >= 1)
            def _():
                scatter_wait()

            scatter_start(t, t % 3)
        return carry

    lax.fori_loop(0, MAINM // 6, body, 0)
    for kk in range(MAINM, NCHM):
        gather_wait()
        if kk + 1 < NCHM:
            idx_wait()
            gather_start((kk + 1) % 6, (kk + 1) % 3)
        if kk + 2 < NCHM:
            idx_start(kk + 2, (kk + 2) % 6)
        compute(kk % 6, kk % 3)
        scatter_wait()
        scatter_start(kk % 6, kk % 3)
    scatter_wait()

    plsc.subcore_barrier()
    for j in range(8):
        ci = sid + j * NS

        @pl.when(ci < N_N // MB)
        def _():
            pltpu.sync_copy(acc_sh.at[pl.ds(ci * MB, MB), :], rows3.at[0])
            pltpu.sync_copy(rows3.at[0],
                            out_ref.at[cid, pl.ds(ci * MB, MB), :])


def _comb_body(p_ref, x_ref, x0_ref, al_ref, ar_ref, dinv_ref, out_ref):
    s = jnp.tanh(al_ref[...] + ar_ref[...]) * dinv_ref[...] * dinv_ref[...]
    out_ref[...] = ((p_ref[0] + p_ref[1]) * dinv_ref[...] + s * x_ref[...]
                    + EPS_V * x0_ref[...])


_comb_call = pl.pallas_call(
    _comb_body,
    out_shape=jax.ShapeDtypeStruct((N_N, C_C), jnp.float32),
)


def kernel(x, x_0, edge_index, test_idx, att_l_w, att_r_w):
    erow = edge_index[0].astype(jnp.int32)
    ecol = edge_index[1].astype(jnp.int32)
    zeros_b = jnp.zeros((B,), jnp.float32)
    zeros80 = jnp.zeros((MB, C_C), jnp.float32)
    degp = _deg_kernel(erow, ecol, zeros_b)
    al, ar, pk = _alar_call(x, att_l_w, att_r_w)
    dinv_col, y = _ds_call(x, degp[:, :N_N].reshape(NC, N_N, 1))
    p = _msg_kernel(erow, ecol, y, pk[:, 0], zeros80)
    return _comb_call(p, x, x_0, al, ar, dinv_col)


# speedup vs baseline: 1.1721x; 1.1721x over previous
import functools

import jax
import jax.numpy as jnp
from jax import lax
from jax.experimental import pallas as pl
from jax.experimental.pallas import tpu as pltpu
from jax.experimental.pallas import tpu_sc as plsc

N_N = 10000
C_C = 128
E_E = 320000
EPS_V = 0.1

NC, NS, L = 2, 16, 16
NW = NC * NS
NP = 10240
EPW = E_E // NW
B = 80
NCH = EPW // B
MAIN = (NCH // 6) * 6
MB = 80
NCHM = EPW // MB
MAINM = (NCHM // 6) * 6

_mesh = plsc.VectorSubcoreMesh(
    core_axis_name="c", subcore_axis_name="s", num_cores=NC, num_subcores=NS)


@functools.partial(
    pl.kernel,
    out_type=jax.ShapeDtypeStruct((NC, NP), jnp.float32),
    mesh=_mesh,
    scratch_types=[
        pltpu.VMEM((EPW,), jnp.int32),
        pltpu.VMEM((NCH, B), jnp.int32),
        pltpu.VMEM((3, B), jnp.float32),
        pltpu.VMEM((B,), jnp.float32),
        pltpu.VMEM((NP // NS,), jnp.float32),
        pltpu.VMEM_SHARED((NP,), jnp.float32),
        pltpu.SemaphoreType.DMA,
        pltpu.SemaphoreType.DMA,
    ],
    compiler_params=pltpu.CompilerParams(needs_layout_passes=False),
)
def _deg_kernel(erow_ref, ecol3_ref, zb_ref, out_ref, rball, cball, vbuf, zb,
                bounce, deg_sh, ssem0, ssem1):
    cid = lax.axis_index("c")
    sid = lax.axis_index("s")
    w = sid * NC + cid
    base = w * EPW

    pltpu.sync_copy(erow_ref.at[pl.ds(pl.multiple_of(base, 8), EPW)], rball)
    pltpu.sync_copy(ecol3_ref.at[w], cball)
    pltpu.sync_copy(zb_ref, zb)
    for j in range(8):
        pltpu.sync_copy(zb, deg_sh.at[pl.ds((sid + j * NS) * B, B)])
    plsc.subcore_barrier()

    def vals(kk, s3):
        for i in range(B // L):
            r = rball[pl.ds(kk * B + i * L, L)]
            c = cball[kk, pl.ds(i * L, L)]
            vbuf[s3, pl.ds(i * L, L)] = jnp.where(r != c, 1.0, 0.0)

    def scatter_start(kk, s3, sem):
        pltpu.async_copy(vbuf.at[s3], deg_sh.at[cball.at[kk]], sem, add=True)

    def scatter_wait(sem):
        pltpu.make_async_copy(vbuf.at[0], deg_sh.at[cball.at[0]], sem).wait()

    def body(k6, carry):
        kb = k6 * 6
        for t in range(6):
            kk = kb + t
            vals(kk, t % 3)
            sem = ssem0 if t % 2 == 0 else ssem1

            @pl.when(kk >= 2)
            def _():
                scatter_wait(sem)

            scatter_start(kk, t % 3, sem)
        return carry

    lax.fori_loop(0, MAIN // 6, body, 0)
    for kk in range(MAIN, NCH):
        vals(kk, kk % 3)
        sem = ssem0 if kk % 2 == 0 else ssem1
        scatter_wait(sem)
        scatter_start(kk, kk % 3, sem)
    scatter_wait(ssem1)
    scatter_wait(ssem0)

    plsc.subcore_barrier()
    rpt = NP // NS
    pltpu.sync_copy(deg_sh.at[pl.ds(sid * rpt, rpt)], bounce)
    pltpu.sync_copy(bounce, out_ref.at[cid, pl.ds(sid * rpt, rpt)])


def _alar_body(x_ref, wl_ref, wr_ref, al_ref, ar_ref, pk_ref):
    x = x_ref[...]
    al = lax.dot_general(x, wl_ref[...], (((1,), (1,)), ((), ())))
    ar = lax.dot_general(x, wr_ref[...], (((1,), (1,)), ((), ())))
    al_ref[...] = al
    ar_ref[...] = ar
    ab = lax.bitcast_convert_type(al, jnp.int32)
    lo = ((ab + 0x7FFF + ((ab >> 16) & 1)) >> 16) & 0xFFFF
    bb = lax.bitcast_convert_type(ar, jnp.int32)
    hi = (bb + 0x7FFF + ((bb >> 16) & 1)) & jnp.int32(-65536)
    pk_ref[...] = hi | lo


_alar_call = pl.pallas_call(
    _alar_body,
    out_shape=(
        jax.ShapeDtypeStruct((N_N, 1), jnp.float32),
        jax.ShapeDtypeStruct((N_N, 1), jnp.float32),
        jax.ShapeDtypeStruct((N_N, 1), jnp.int32),
    ),
)


def _ds_body(x_ref, degp_ref, dinv_ref, y_ref):
    dinv = lax.rsqrt(degp_ref[0] + degp_ref[1] + 1.0)
    dinv_ref[...] = dinv
    y_ref[...] = x_ref[...] * dinv


_ds_call = pl.pallas_call(
    _ds_body,
    out_shape=(
        jax.ShapeDtypeStruct((N_N, 1), jnp.float32),
        jax.ShapeDtypeStruct((N_N, C_C), jnp.float32),
    ),
)


@functools.partial(
    pl.kernel,
    out_type=jax.ShapeDtypeStruct((NC, N_N, C_C), jnp.float32),
    mesh=_mesh,
    scratch_types=[
        pltpu.VMEM((N_N,), jnp.int32),
        pltpu.VMEM((6, MB), jnp.int32),
        pltpu.VMEM((6, MB), jnp.int32),
        pltpu.VMEM((3, MB, C_C), jnp.float32),
        pltpu.VMEM_SHARED((N_N, C_C), jnp.float32),
        pltpu.SemaphoreType.DMA,
        pltpu.SemaphoreType.DMA,
        pltpu.SemaphoreType.DMA,
    ],
    compiler_params=pltpu.CompilerParams(needs_layout_passes=False),
)
def _msg_kernel(erow_ref, ecol_ref, y_ref, pk_ref, zb_ref,
                out_ref, pkv, rbuf, cbuf, rows3, acc_sh,
                isem, gsem, ssem):
    cid = lax.axis_index("c")
    sid = lax.axis_index("s")
    w = sid * NC + cid
    base = w * EPW

    pltpu.sync_copy(pk_ref, pkv)
    pltpu.sync_copy(zb_ref, rows3.at[0])
    for j in range(8):
        ci = sid + j * NS

        @pl.when(ci < N_N // MB)
        def _():
            pltpu.sync_copy(rows3.at[0], acc_sh.at[pl.ds(ci * MB, MB), :])

    plsc.subcore_barrier()

    def idx_start(off_chunks, s6):
        off = pl.multiple_of(base + off_chunks * MB, 8)
        pltpu.async_copy(erow_ref.at[pl.ds(off, MB)], rbuf.at[s6], isem)
        pltpu.async_copy(ecol_ref.at[pl.ds(off, MB)], cbuf.at[s6], isem)

    def idx_wait():
        pltpu.make_async_copy(erow_ref.at[pl.ds(0, MB)], rbuf.at[0],
                              isem).wait()
        pltpu.make_async_copy(ecol_ref.at[pl.ds(0, MB)], cbuf.at[0],
                              isem).wait()

    def gather_start(s6, s3):
        pltpu.async_copy(y_ref.at[rbuf.at[s6]], rows3.at[s3], gsem)

    def gather_wait():
        pltpu.make_async_copy(y_ref.at[rbuf.at[0]], rows3.at[0], gsem).wait()

    def scatter_start(s6, s3):
        pltpu.async_copy(rows3.at[s3], acc_sh.at[cbuf.at[s6]], ssem, add=True)

    def scatter_wait():
        pltpu.make_async_copy(rows3.at[0], acc_sh.at[cbuf.at[0]],
                              ssem).wait()

    def compute(s6, s3):
        def grp(i, carry):
            r = rbuf[s6, pl.ds(i * L, L)]
            c = cbuf[s6, pl.ds(i * L, L)]
            pr = plsc.load_gather(pkv, [r])
            pc = plsc.load_gather(pkv, [c])
            af = plsc.bitcast(pr << 16, jnp.float32)
            bf = plsc.bitcast(pc & jnp.int32(-65536), jnp.float32)
            t = 1.0 - 2.0 / (jnp.exp(2.0 * (af + bf)) + 1.0)
            cv = jnp.where(r != c, t, 0.0)
            for e in range(L):
                cs = cv[e]
                row = i * L + e
                for j in range(C_C // L):
                    rows3[s3, row, pl.ds(j * L, L)] = (
                        rows3[s3, row, pl.ds(j * L, L)] * cs)
            return carry

        lax.fori_loop(0, MB // L, grp, 0)

    idx_start(0, 0)
    idx_wait()
    gather_start(0, 0)
    idx_start(1, 1)

    def body(k6, carry):
        kb = k6 * 6
        for t in range(6):
            kk = kb + t
            gather_wait()
            idx_wait()
            gather_start((t + 1) % 6, (t + 1) % 3)
            idx_start(kk + 2, (t + 2) % 6)
            compute(t, t % 3)

            @pl.when(kk >= 1)
            def _():
                scatter_wait()

            scatter_start(t, t % 3)
        return carry

    lax.fori_loop(0, MAINM // 6, body, 0)
    for kk in range(MAINM, NCHM):
        gather_wait()
        if kk + 1 < NCHM:
            idx_wait()
            gather_start((kk + 1) % 6, (kk + 1) % 3)
        if kk + 2 < NCHM:
            idx_start(kk + 2, (kk + 2) % 6)
        compute(kk % 6, kk % 3)
        scatter_wait()
        scatter_start(kk % 6, kk % 3)
    scatter_wait()

    plsc.subcore_barrier()
    for j in range(8):
        ci = sid + j * NS

        @pl.when(ci < N_N // MB)
        def _():
            pltpu.sync_copy(acc_sh.at[pl.ds(ci * MB, MB), :], rows3.at[0])
            pltpu.sync_copy(rows3.at[0],
                            out_ref.at[cid, pl.ds(ci * MB, MB), :])


def _comb_body(p_ref, x_ref, x0_ref, al_ref, ar_ref, dinv_ref, out_ref):
    s = jnp.tanh(al_ref[...] + ar_ref[...]) * dinv_ref[...] * dinv_ref[...]
    out_ref[...] = ((p_ref[0] + p_ref[1]) * dinv_ref[...] + s * x_ref[...]
                    + EPS_V * x0_ref[...])


_comb_call = pl.pallas_call(
    _comb_body,
    out_shape=jax.ShapeDtypeStruct((N_N, C_C), jnp.float32),
)


def kernel(x, x_0, edge_index, test_idx, att_l_w, att_r_w):
    erow = edge_index[0].astype(jnp.int32)
    ecol = edge_index[1].astype(jnp.int32)
    zeros_b = jnp.zeros((B,), jnp.float32)
    zeros80 = jnp.zeros((MB, C_C), jnp.float32)
    ecol3 = ecol.reshape(NW, NCH, B)
    degp = _deg_kernel(erow, ecol3, zeros_b)
    al, ar, pk = _alar_call(x, att_l_w, att_r_w)
    dinv_col, y = _ds_call(x, degp[:, :N_N].reshape(NC, N_N, 1))
    p = _msg_kernel(erow, ecol, y, pk[:, 0], zeros80)
    return _comb_call(p, x, x_0, al, ar, dinv_col)
